# baseline (speedup 1.0000x reference)
import jax
import jax.numpy as jnp
from jax.experimental import pallas as pl
from jax.experimental.pallas import tpu as pltpu

_N = 10000
_S1 = 4096
_B = 1024
_F = 128
_H = 128
_C = 64

_KCH = 400
_NKCH = _N // _KCH
_NSLOT = 3

_XCH = 2000
_A2CH = 128

_PREC = jax.lax.Precision.DEFAULT


_NSUB = 2
_SUBR = _KCH // _NSUB


def _a1_copies(a1t_ref, ring_ref, sem_ref, g, slot):
    return [pltpu.make_async_copy(
        a1t_ref.at[pl.ds(g * _KCH + j * _SUBR, _SUBR), :],
        ring_ref.at[slot, pl.ds(j * _SUBR, _SUBR), :],
        sem_ref.at[slot]) for j in range(_NSUB)]


def _gcn_kernel(x_ref, w1_ref, b1_ref, a1t_ref, a2_ref, w2_ref, b2_ref,
                out_ref, ring_ref, x_vmem, a2_vmem, s1t_ref, acc_ref,
                a1_sem, x_sem, a2_sem):
    for c in range(_N // _XCH):
        pltpu.make_async_copy(
            x_ref.at[pl.ds(c * _XCH, _XCH), :],
            x_vmem.at[pl.ds(c * _XCH, _XCH), :], x_sem).start()
    for g in range(_NSLOT):
        for cp in _a1_copies(a1t_ref, ring_ref, a1_sem, g, g):
            cp.start()
    for c in range(_B // _A2CH):
        pltpu.make_async_copy(
            a2_ref.at[pl.ds(c * _A2CH, _A2CH), :],
            a2_vmem.at[pl.ds(c * _A2CH, _A2CH), :], a2_sem).start()
    for c in range(_N // _XCH):
        pltpu.make_async_copy(
            x_ref.at[pl.ds(c * _XCH, _XCH), :],
            x_vmem.at[pl.ds(c * _XCH, _XCH), :], x_sem).wait()

    s1 = jax.lax.dot_general(
        x_vmem[...], w1_ref[...], (((1,), (0,)), ((), ())),
        precision=_PREC, preferred_element_type=jnp.float32)
    for gg in range(_NKCH):
        s1t_ref[gg] = jnp.transpose(
            s1[gg * _KCH:(gg + 1) * _KCH, :]).astype(jnp.bfloat16)

    def body(g, carry):
        slot = jax.lax.rem(g, _NSLOT)
        for cp in _a1_copies(a1t_ref, ring_ref, a1_sem, g, slot):
            cp.wait()
        a_bf = ring_ref[slot].astype(jnp.bfloat16)
        p = jax.lax.dot_general(
            s1t_ref[g], a_bf, (((1,), (0,)), ((), ())),
            precision=_PREC, preferred_element_type=jnp.float32)

        @pl.when(g == 0)
        def _():
            acc_ref[...] = p

        @pl.when(g > 0)
        def _():
            acc_ref[...] = acc_ref[...] + p

        @pl.when(g + _NSLOT < _NKCH)
        def _():
            for cp in _a1_copies(a1t_ref, ring_ref, a1_sem, g + _NSLOT, slot):
                cp.start()
        return carry

    jax.lax.fori_loop(0, _NKCH, body, 0)

    h = jnp.transpose(jnp.maximum(acc_ref[...] + b1_ref[...], 0.0))
    s2 = jax.lax.dot_general(
        h, w2_ref[...], (((1,), (0,)), ((), ())),
        precision=_PREC, preferred_element_type=jnp.float32)
    for c in range(_B // _A2CH):
        pltpu.make_async_copy(
            a2_ref.at[pl.ds(c * _A2CH, _A2CH), :],
            a2_vmem.at[pl.ds(c * _A2CH, _A2CH), :], a2_sem).wait()
    logits = jax.lax.dot_general(
        a2_vmem[...], s2, (((1,), (0,)), ((), ())),
        precision=_PREC, preferred_element_type=jnp.float32) + b2_ref[...]
    m = jnp.max(logits, axis=1, keepdims=True)
    lse = jnp.log(jnp.sum(jnp.exp(logits - m), axis=1, keepdims=True)) + m
    out_ref[...] = logits - lse


def kernel(feature, adj1, adj2, W1, b1, W2, b2):
    a1t = adj1.T
    b1r = b1.reshape(_H, 1)
    b2r = b2.reshape(1, _C)

    hbm = pl.BlockSpec(memory_space=pltpu.MemorySpace.HBM)
    return pl.pallas_call(
        _gcn_kernel,
        in_specs=[
            hbm,
            pl.BlockSpec((_F, _H), lambda: (0, 0)),
            pl.BlockSpec((_H, 1), lambda: (0, 0)),
            hbm,
            hbm,
            pl.BlockSpec((_H, _C), lambda: (0, 0)),
            pl.BlockSpec((1, _C), lambda: (0, 0)),
        ],
        out_specs=pl.BlockSpec((_B, _C), lambda: (0, 0)),
        out_shape=jax.ShapeDtypeStruct((_B, _C), jnp.float32),
        scratch_shapes=[
            pltpu.VMEM((_NSLOT, _KCH, _S1), jnp.float32),
            pltpu.VMEM((_N, _F), jnp.float32),
            pltpu.VMEM((_B, _S1), jnp.float32),
            pltpu.VMEM((_NKCH, _H, _KCH), jnp.bfloat16),
            pltpu.VMEM((_H, _S1), jnp.float32),
            pltpu.SemaphoreType.DMA((_NSLOT,)),
            pltpu.SemaphoreType.DMA,
            pltpu.SemaphoreType.DMA,
        ],
        compiler_params=pltpu.CompilerParams(
            vmem_limit_bytes=100 * 1024 * 1024),
    )(feature, W1, b1r, a1t, adj2, W2, b2r)


# speedup vs baseline: 1.0889x; 1.0889x over previous
import jax
import jax.numpy as jnp
from jax.experimental import pallas as pl
from jax.experimental.pallas import tpu as pltpu

_N = 10000
_S1 = 4096
_B = 1024
_F = 128
_H = 128
_C = 64

_KCH = 400
_NKCH = _N // _KCH
_NSLOT = 3

_XCH = 2000
_A2CH = 128

_PREC = jax.lax.Precision.DEFAULT


_NSUB = 1
_SUBR = _KCH // _NSUB


def _a1_copies(a1t_ref, ring_ref, sem_ref, g, slot):
    return [pltpu.make_async_copy(
        a1t_ref.at[pl.ds(g * _KCH + j * _SUBR, _SUBR), :],
        ring_ref.at[slot, pl.ds(j * _SUBR, _SUBR), :],
        sem_ref.at[slot]) for j in range(_NSUB)]


def _gcn_kernel(x_ref, w1_ref, b1_ref, a1t_ref, a2_ref, w2t_ref, b2_ref,
                out_ref, ring_ref, x_vmem, a2_vmem, s1t_ref, acc_ref,
                a1_sem, x_sem, a2_sem):
    for c in range(_N // _XCH):
        pltpu.make_async_copy(
            x_ref.at[pl.ds(c * _XCH, _XCH), :],
            x_vmem.at[pl.ds(c * _XCH, _XCH), :], x_sem).start()
    for g in range(_NSLOT):
        for cp in _a1_copies(a1t_ref, ring_ref, a1_sem, g, g):
            cp.start()
    for c in range(_B // _A2CH):
        pltpu.make_async_copy(
            a2_ref.at[pl.ds(c * _A2CH, _A2CH), :],
            a2_vmem.at[pl.ds(c * _A2CH, _A2CH), :], a2_sem).start()
    for c in range(_N // _XCH):
        pltpu.make_async_copy(
            x_ref.at[pl.ds(c * _XCH, _XCH), :],
            x_vmem.at[pl.ds(c * _XCH, _XCH), :], x_sem).wait()

    s1 = jax.lax.dot_general(
        x_vmem[...], w1_ref[...], (((1,), (0,)), ((), ())),
        precision=_PREC, preferred_element_type=jnp.float32)
    for gg in range(_NKCH):
        s1t_ref[gg] = jnp.transpose(
            s1[gg * _KCH:(gg + 1) * _KCH, :]).astype(jnp.bfloat16)

    def body(g, carry):
        slot = jax.lax.rem(g, _NSLOT)
        for cp in _a1_copies(a1t_ref, ring_ref, a1_sem, g, slot):
            cp.wait()
        a_bf = ring_ref[slot].astype(jnp.bfloat16)
        p = jax.lax.dot_general(
            s1t_ref[g], a_bf, (((1,), (0,)), ((), ())),
            precision=_PREC, preferred_element_type=jnp.float32)

        @pl.when(g == 0)
        def _():
            acc_ref[...] = p

        @pl.when(g > 0)
        def _():
            acc_ref[...] = acc_ref[...] + p

        @pl.when(g + _NSLOT < _NKCH)
        def _():
            for cp in _a1_copies(a1t_ref, ring_ref, a1_sem, g + _NSLOT, slot):
                cp.start()
        return carry

    jax.lax.fori_loop(0, _NKCH, body, 0)

    b1col = jnp.transpose(b1_ref[...])
    h = jnp.transpose(jnp.maximum(acc_ref[...] + b1col, 0.0))
    s2 = jax.lax.dot_general(
        h, w2t_ref[...], (((1,), (1,)), ((), ())),
        precision=_PREC, preferred_element_type=jnp.float32)
    for c in range(_B // _A2CH):
        pltpu.make_async_copy(
            a2_ref.at[pl.ds(c * _A2CH, _A2CH), :],
            a2_vmem.at[pl.ds(c * _A2CH, _A2CH), :], a2_sem).wait()
    logits = jax.lax.dot_general(
        a2_vmem[...], s2, (((1,), (0,)), ((), ())),
        precision=_PREC, preferred_element_type=jnp.float32) + b2_ref[...]
    m = jnp.max(logits, axis=1, keepdims=True)
    lse = jnp.log(jnp.sum(jnp.exp(logits - m), axis=1, keepdims=True)) + m
    out_ref[...] = jnp.transpose(logits - lse)


def kernel(feature, adj1, adj2, W1, b1, W2, b2):
    a1t = adj1.T
    w2t = W2.T
    b1r = b1.reshape(1, _H)
    b2r = b2.reshape(1, _C)

    hbm = pl.BlockSpec(memory_space=pltpu.MemorySpace.HBM)
    out_t = pl.pallas_call(
        _gcn_kernel,
        in_specs=[
            hbm,
            pl.BlockSpec((_F, _H), lambda: (0, 0)),
            pl.BlockSpec((1, _H), lambda: (0, 0)),
            hbm,
            hbm,
            pl.BlockSpec((_C, _H), lambda: (0, 0)),
            pl.BlockSpec((1, _C), lambda: (0, 0)),
        ],
        out_specs=pl.BlockSpec((_C, _B), lambda: (0, 0)),
        out_shape=jax.ShapeDtypeStruct((_C, _B), jnp.float32),
        scratch_shapes=[
            pltpu.VMEM((_NSLOT, _KCH, _S1), jnp.float32),
            pltpu.VMEM((_N, _F), jnp.float32),
            pltpu.VMEM((_B, _S1), jnp.float32),
            pltpu.VMEM((_NKCH, _H, _KCH), jnp.bfloat16),
            pltpu.VMEM((_H, _S1), jnp.float32),
            pltpu.SemaphoreType.DMA((_NSLOT,)),
            pltpu.SemaphoreType.DMA,
            pltpu.SemaphoreType.DMA,
        ],
        compiler_params=pltpu.CompilerParams(
            vmem_limit_bytes=100 * 1024 * 1024),
    )(feature, W1, b1r, a1t, adj2, w2t, b2r)
    return out_t.T
